# Initial kernel scaffold; baseline (speedup 1.0000x reference)
import jax
import jax.numpy as jnp
from jax.experimental import pallas as pl

_NE = 512
_D = 32
_B = 128
_T = 1024
_ROWS = _B * _T
_BLK = 2048


def _vq_kernel(xt_ref, xraw_ref, embT_ref, emb_ref, e2_ref, out_ref, loss_ref):
    i = pl.program_id(0)
    xt = xt_ref[...]
    scores = jnp.dot(xt, embT_ref[...], preferred_element_type=jnp.float32)
    m = e2_ref[...] - 2.0 * scores
    minval = jnp.min(m, axis=1, keepdims=True)
    iota = jax.lax.broadcasted_iota(jnp.int32, m.shape, 1)
    idx = jnp.min(jnp.where(m == minval, iota, _NE), axis=1, keepdims=True)
    onehot = jnp.where(iota == idx, 1.0, 0.0)
    gathered = jnp.dot(onehot, emb_ref[...], preferred_element_type=jnp.float32)
    out_ref[...] = gathered
    diff = gathered - xraw_ref[...]
    psum = jnp.sum(diff * diff)

    @pl.when(i == 0)
    def _():
        loss_ref[...] = jnp.zeros_like(loss_ref)

    loss_ref[...] += psum


@jax.jit
def _vq(x, embeddings):
    xt = jnp.transpose(x, (0, 2, 1)).reshape(_ROWS, _D)
    xraw = x.reshape(_ROWS, _D)
    embT = embeddings.T
    e2 = jnp.sum(embeddings * embeddings, axis=1)[None, :]
    grid = _ROWS // _BLK
    out, losssum = pl.pallas_call(
        _vq_kernel,
        grid=(grid,),
        in_specs=[
            pl.BlockSpec((_BLK, _D), lambda i: (i, 0)),
            pl.BlockSpec((_BLK, _D), lambda i: (i, 0)),
            pl.BlockSpec((_D, _NE), lambda i: (0, 0)),
            pl.BlockSpec((_NE, _D), lambda i: (0, 0)),
            pl.BlockSpec((1, _NE), lambda i: (0, 0)),
        ],
        out_specs=[
            pl.BlockSpec((_BLK, _D), lambda i: (i, 0)),
            pl.BlockSpec((1, 1), lambda i: (0, 0)),
        ],
        out_shape=[
            jax.ShapeDtypeStruct((_ROWS, _D), jnp.float32),
            jax.ShapeDtypeStruct((1, 1), jnp.float32),
        ],
    )(xt, xraw, embT, embeddings, e2)
    quantized = out.reshape(x.shape)
    loss = losssum[0, 0] * (1.25 / x.size)
    return quantized, loss


def kernel(x, embeddings):
    return _vq(x, embeddings)


# speedup vs baseline: 2.5629x; 2.5629x over previous
import jax
import jax.numpy as jnp
from jax.experimental import pallas as pl

_NE = 512
_D = 32
_B = 128
_T = 1024
_ROWS = _B * _T
_BLK = 2048


def _vq_kernel(xt_ref, xraw_ref, embT_ref, emb_ref, e2_ref, out_ref, loss_ref):
    i = pl.program_id(0)
    xt = xt_ref[...]
    scores = jnp.dot(xt, embT_ref[...], preferred_element_type=jnp.float32)
    x2 = jnp.sum(xt * xt, axis=1, keepdims=True)
    m = (x2 + e2_ref[...]) - 2.0 * scores
    minval = jnp.min(m, axis=1, keepdims=True)
    iota = jax.lax.broadcasted_iota(jnp.int32, m.shape, 1)
    idx = jnp.min(jnp.where(m == minval, iota, _NE), axis=1, keepdims=True)
    onehot = jnp.where(iota == idx, 1.0, 0.0)
    gathered = jnp.dot(onehot, emb_ref[...], preferred_element_type=jnp.float32)
    out_ref[...] = gathered
    diff = gathered - xraw_ref[...]
    psum = jnp.sum(diff * diff)

    @pl.when(i == 0)
    def _():
        loss_ref[...] = jnp.zeros_like(loss_ref)

    loss_ref[...] += psum


@jax.jit
def _vq(x, embeddings):
    xt = jnp.transpose(x, (0, 2, 1)).reshape(_ROWS, _D)
    xraw = x.reshape(_ROWS, _D)
    embT = embeddings.T
    e2 = jnp.sum(embeddings * embeddings, axis=1)[None, :]
    grid = _ROWS // _BLK
    out, losssum = pl.pallas_call(
        _vq_kernel,
        grid=(grid,),
        in_specs=[
            pl.BlockSpec((_BLK, _D), lambda i: (i, 0)),
            pl.BlockSpec((_BLK, _D), lambda i: (i, 0)),
            pl.BlockSpec((_D, _NE), lambda i: (0, 0)),
            pl.BlockSpec((_NE, _D), lambda i: (0, 0)),
            pl.BlockSpec((1, _NE), lambda i: (0, 0)),
        ],
        out_specs=[
            pl.BlockSpec((_BLK, _D), lambda i: (i, 0)),
            pl.BlockSpec((1, 1), lambda i: (0, 0)),
        ],
        out_shape=[
            jax.ShapeDtypeStruct((_ROWS, _D), jnp.float32),
            jax.ShapeDtypeStruct((1, 1), jnp.float32),
        ],
    )(xt, xraw, embT, embeddings, e2)
    quantized = out.reshape(x.shape)
    loss = losssum[0, 0] * (1.25 / x.size)
    return quantized, loss


def kernel(x, embeddings):
    return _vq(x, embeddings)
